# baseline (speedup 1.0000x reference)
import functools

import jax
import jax.numpy as jnp
from jax import lax
from jax.experimental import pallas as pl
from jax.experimental.pallas import tpu as pltpu
from jax.experimental.pallas import tpu_sc as plsc

_INFO = plsc.get_sparse_core_info()
_NC, _NS = _INFO.num_cores, _INFO.num_subcores
_NW = _NC * _NS

_C = 4
_NBUF = 3


@jax.jit
def _gather_rows(idx2, table):
    n_rows_total, c = idx2.shape
    N = n_rows_total * c
    V, D = table.shape
    b_per_w = N // _NW
    n_chunks = b_per_w // _C
    n_rounds = n_chunks // _NBUF + 1
    mesh = plsc.VectorSubcoreMesh(core_axis_name="c", subcore_axis_name="s")

    @functools.partial(
        pl.kernel,
        mesh=mesh,
        out_type=jax.ShapeDtypeStruct((N, D), jnp.float32),
        scratch_types=[
            pltpu.VMEM((n_chunks, _C), jnp.int32),
            pltpu.VMEM((_NBUF, _C, D), jnp.float32),
            pltpu.SemaphoreType.DMA,
            pltpu.SemaphoreType.DMA,
            pltpu.SemaphoreType.DMA,
            pltpu.SemaphoreType.DMA,
            pltpu.SemaphoreType.DMA,
            pltpu.SemaphoreType.DMA,
        ],
    )
    def body(idx_hbm, table_hbm, out_hbm, idx_v, rows_v, g0, g1, g2, s0, s1, s2):
        gsem = (g0, g1, g2)
        ssem = (s0, s1, s2)
        wid = lax.axis_index("s") * _NC + lax.axis_index("c")
        base = wid * b_per_w
        pltpu.sync_copy(idx_hbm.at[pl.ds(wid * n_chunks, n_chunks), :], idx_v)

        def gather_copy(ch, b):
            return pltpu.make_async_copy(
                table_hbm.at[idx_v.at[ch]], rows_v.at[b], gsem[b])

        def scatter_copy(ch, b):
            return pltpu.make_async_copy(
                rows_v.at[b], out_hbm.at[pl.ds(base + ch * _C, _C)], ssem[b])

        gather_copy(0, 0).start()

        def round_(s, carry):
            for b in range(_NBUF):
                ch = s * _NBUF + b
                bn = (b + 1) % _NBUF

                @pl.when(jnp.logical_and(ch >= 2, ch - 2 < n_chunks))
                def _():
                    scatter_copy(ch - 2, bn).wait()

                @pl.when(ch + 1 < n_chunks)
                def _():
                    gather_copy(ch + 1, bn).start()

                @pl.when(ch < n_chunks)
                def _():
                    gather_copy(ch, b).wait()
                    scatter_copy(ch, b).start()
            return carry

        lax.fori_loop(0, n_rounds, round_, 0)

    return body(idx2, table)


def kernel(X, table):
    B, T = X.shape
    idx2 = X.reshape(B * T // _C, _C).astype(jnp.int32)
    out = _gather_rows(idx2, table)
    return out.reshape(B, T, table.shape[1])


# speedup vs baseline: 1.0020x; 1.0020x over previous
import functools

import jax
import jax.numpy as jnp
from jax import lax
from jax.experimental import pallas as pl
from jax.experimental.pallas import tpu as pltpu
from jax.experimental.pallas import tpu_sc as plsc

_INFO = plsc.get_sparse_core_info()
_NC, _NS = _INFO.num_cores, _INFO.num_subcores
_NW = _NC * _NS

_C = 4
_NBUF = 3


@jax.jit
def _gather_rows(idx2, table):
    n_rows_total, c = idx2.shape
    N = n_rows_total * c
    V, D = table.shape
    b_per_w = N // _NW
    n_chunks = b_per_w // _C
    n_rounds = n_chunks // _NBUF + 1
    mesh = plsc.VectorSubcoreMesh(core_axis_name="c", subcore_axis_name="s")

    @functools.partial(
        pl.kernel,
        mesh=mesh,
        out_type=jax.ShapeDtypeStruct((N, D), jnp.float32),
        scratch_types=[
            pltpu.VMEM((n_chunks, _C), jnp.int32),
            pltpu.VMEM((_NBUF, _C, D), jnp.float32),
        ] + [pltpu.SemaphoreType.DMA] * (2 * _NBUF),
    )
    def body(idx_hbm, table_hbm, out_hbm, idx_v, rows_v, *sems):
        gsem = sems[:_NBUF]
        ssem = sems[_NBUF:]
        wid = lax.axis_index("s") * _NC + lax.axis_index("c")
        base = wid * b_per_w
        pltpu.sync_copy(idx_hbm.at[pl.ds(wid * n_chunks, n_chunks), :], idx_v)

        def gather_copy(ch, b):
            return pltpu.make_async_copy(
                table_hbm.at[idx_v.at[ch]], rows_v.at[b], gsem[b])

        def scatter_copy(ch, b):
            return pltpu.make_async_copy(
                rows_v.at[b], out_hbm.at[pl.ds(base + ch * _C, _C)], ssem[b])

        gather_copy(0, 0).start()

        def round_(s, carry):
            for b in range(_NBUF):
                ch = s * _NBUF + b
                bn = (b + 1) % _NBUF

                @pl.when(jnp.logical_and(ch >= _NBUF - 1,
                                         ch - (_NBUF - 1) < n_chunks))
                def _():
                    scatter_copy(ch - (_NBUF - 1), bn).wait()

                @pl.when(ch + 1 < n_chunks)
                def _():
                    gather_copy(ch + 1, bn).start()

                @pl.when(ch < n_chunks)
                def _():
                    gather_copy(ch, b).wait()
                    scatter_copy(ch, b).start()
            return carry

        lax.fori_loop(0, n_rounds, round_, 0)

    return body(idx2, table)


def kernel(X, table):
    B, T = X.shape
    idx2 = X.reshape(B * T // _C, _C).astype(jnp.int32)
    out = _gather_rows(idx2, table)
    return out.reshape(B, T, table.shape[1])
